# baseline (speedup 1.0000x reference)
import functools

import jax
import jax.numpy as jnp
from jax import lax
from jax.experimental import pallas as pl
from jax.experimental.pallas import tpu as pltpu
from jax.experimental.pallas import tpu_sc as plsc

NPOS = 8192
NPRIM = 1024
NORB = 256
BN = 1024

NSC = 1024
NTC = NPOS - NSC
NWORK = 32
RPW = NSC // NWORK
L = 16
BSC = 512

_LOG2E = 1.4426950408889634


def _basis_block(pos_ref, cn_ref, centerT_ref, at_ref, lmnT_ref, oi_ref,
                 out_ref):
    p = pos_ref[...]
    x = p[:, 0:1]
    y = p[:, 1:2]
    z = p[:, 2:3]

    cx = centerT_ref[0:1, :]
    cy = centerT_ref[1:2, :]
    cz = centerT_ref[2:3, :]

    dx = x - cx
    dy = y - cy
    dz = z - cz
    d2x = dx * dx
    d2y = dy * dy
    d2z = dz * dz
    r2 = (d2x + d2y) + d2z

    lx = lmnT_ref[0:1, :]
    ly = lmnT_ref[1:2, :]
    lz = lmnT_ref[2:3, :]
    ax = jnp.where(lx == 0, 1.0, jnp.where(lx == 1, dx, d2x))
    ay = jnp.where(ly == 0, 1.0, jnp.where(ly == 1, dy, d2y))
    az = jnp.where(lz == 0, 1.0, jnp.where(lz == 1, dz, d2z))

    ex = jnp.exp2(at_ref[...] * r2)
    prim = (cn_ref[...] * ax) * (ay * az) * ex

    col = jax.lax.broadcasted_iota(jnp.int32, (NORB, NPRIM), 0)
    s = (col == oi_ref[...]).astype(jnp.float32)
    out_ref[...] = jax.lax.dot_general(
        prim, s, (((1,), (1,)), ((), ())),
        preferred_element_type=jnp.float32)


def _tc_part(pos, cn, centerT, at, lmnT, oi):
    grid = (NTC // BN,)
    return pl.pallas_call(
        _basis_block,
        grid=grid,
        in_specs=[
            pl.BlockSpec((BN, 3), lambda i: (i, 0)),
            pl.BlockSpec((1, NPRIM), lambda i: (0, 0)),
            pl.BlockSpec((3, NPRIM), lambda i: (0, 0)),
            pl.BlockSpec((1, NPRIM), lambda i: (0, 0)),
            pl.BlockSpec((3, NPRIM), lambda i: (0, 0)),
            pl.BlockSpec((1, NPRIM), lambda i: (0, 0)),
        ],
        out_specs=pl.BlockSpec((BN, NORB), lambda i: (i, 0)),
        out_shape=jax.ShapeDtypeStruct((NTC, NORB), jnp.float32),
        compiler_params=pltpu.CompilerParams(
            dimension_semantics=("parallel",)),
    )(pos, cn, centerT, at, lmnT, oi)


def _sc_body(pxr_hbm, pf_hbm, li_hbm, out_hbm, pxr_v, pf_v, li_v, prim_v):
    wid = lax.axis_index("s") * 2 + lax.axis_index("c")
    base = wid * RPW

    pltpu.sync_copy(pxr_hbm.at[pl.ds(base * 3 * L, RPW * 3 * L)], pxr_v)
    pltpu.sync_copy(pf_hbm, pf_v)
    pltpu.sync_copy(li_hbm, li_v)

    def row_body(r, _):
        x = pxr_v[pl.ds(r * 3 * L, L)]
        y = pxr_v[pl.ds(r * 3 * L + L, L)]
        z = pxr_v[pl.ds(r * 3 * L + 2 * L, L)]

        def slice_body(j, __):
            o = j * L
            cx = pf_v[pl.ds(o, L)]
            cy = pf_v[pl.ds(NPRIM + o, L)]
            cz = pf_v[pl.ds(2 * NPRIM + o, L)]
            at = pf_v[pl.ds(3 * NPRIM + o, L)]
            cn = pf_v[pl.ds(4 * NPRIM + o, L)]
            lx = li_v[pl.ds(o, L)]
            ly = li_v[pl.ds(NPRIM + o, L)]
            lz = li_v[pl.ds(2 * NPRIM + o, L)]
            dx = x - cx
            dy = y - cy
            dz = z - cz
            d2x = dx * dx
            d2y = dy * dy
            d2z = dz * dz
            r2 = (d2x + d2y) + d2z
            ax = jnp.where(lx == 0, 1.0, jnp.where(lx == 1, dx, d2x))
            ay = jnp.where(ly == 0, 1.0, jnp.where(ly == 1, dy, d2y))
            az = jnp.where(lz == 0, 1.0, jnp.where(lz == 1, dz, d2z))
            prim = (cn * ax) * (ay * az) * jnp.exp(at * r2)
            prim_v[pl.ds(r * NPRIM + o, L)] = prim
            return __

        lax.fori_loop(0, NPRIM // L, slice_body, 0, unroll=False)
        return _

    lax.fori_loop(0, RPW, row_body, 0, unroll=False)
    pltpu.sync_copy(prim_v, out_hbm.at[pl.ds(base * NPRIM, RPW * NPRIM)])


def _sc_part(pxr, pf, li):
    mesh = plsc.VectorSubcoreMesh(core_axis_name="c", subcore_axis_name="s")
    run = functools.partial(
        pl.kernel,
        mesh=mesh,
        out_type=jax.ShapeDtypeStruct((NSC * NPRIM,), jnp.float32),
        scratch_types=[
            pltpu.VMEM((RPW * 3 * L,), jnp.float32),
            pltpu.VMEM((5 * NPRIM,), jnp.float32),
            pltpu.VMEM((3 * NPRIM,), jnp.int32),
            pltpu.VMEM((RPW * NPRIM,), jnp.float32),
        ],
    )(_sc_body)
    return run(pxr, pf, li)


def _seg_block(prim_ref, oi_ref, out_ref):
    col = jax.lax.broadcasted_iota(jnp.int32, (NORB, NPRIM), 0)
    s = (col == oi_ref[...]).astype(jnp.float32)
    out_ref[...] = jax.lax.dot_general(
        prim_ref[...], s, (((1,), (1,)), ((), ())),
        preferred_element_type=jnp.float32)


def _seg_part(prim_sc, oi):
    return pl.pallas_call(
        _seg_block,
        grid=(NSC // BSC,),
        in_specs=[
            pl.BlockSpec((BSC, NPRIM), lambda i: (i, 0)),
            pl.BlockSpec((1, NPRIM), lambda i: (0, 0)),
        ],
        out_specs=pl.BlockSpec((BSC, NORB), lambda i: (i, 0)),
        out_shape=jax.ShapeDtypeStruct((NSC, NORB), jnp.float32),
        compiler_params=pltpu.CompilerParams(
            dimension_semantics=("parallel",)),
    )(prim_sc, oi)


@jax.jit
def kernel(pos, coefficients, center, alpha, norm, lmn, orbital_index):
    cn = (coefficients * norm).reshape(1, NPRIM)
    centerT = center.T
    lmnT = lmn.T
    at = (-_LOG2E * alpha).reshape(1, NPRIM)
    oi = orbital_index.reshape(1, NPRIM)

    out_tc = _tc_part(pos[:NTC], cn, centerT, at, lmnT, oi)

    pxr = jnp.repeat(pos[NTC:, :, None], L, axis=2).reshape(-1)
    pf = jnp.concatenate(
        [center[:, 0], center[:, 1], center[:, 2], -alpha,
         (coefficients * norm)], axis=0)
    li = lmnT.reshape(-1)
    prim_sc = _sc_part(pxr, pf, li).reshape(NSC, NPRIM)
    out_sc = _seg_part(prim_sc, oi)

    return jnp.concatenate([out_tc, out_sc], axis=0)


# speedup vs baseline: 1.2402x; 1.2402x over previous
import functools

import jax
import jax.numpy as jnp
from jax import lax
from jax.experimental import pallas as pl
from jax.experimental.pallas import tpu as pltpu
from jax.experimental.pallas import tpu_sc as plsc

NPOS = 8192
NPRIM = 1024
NORB = 256
BN = 512

NSC = 512
NTC = NPOS - NSC
NWORK = 32
RPW = NSC // NWORK
L = 16
BSC = 256

_LOG2E = 1.4426950408889634


def _basis_block(pos_ref, cn_ref, centerT_ref, at_ref, lmnT_ref, oi_ref,
                 out_ref):
    p = pos_ref[...]
    x = p[:, 0:1]
    y = p[:, 1:2]
    z = p[:, 2:3]

    cx = centerT_ref[0:1, :]
    cy = centerT_ref[1:2, :]
    cz = centerT_ref[2:3, :]

    dx = x - cx
    dy = y - cy
    dz = z - cz
    d2x = dx * dx
    d2y = dy * dy
    d2z = dz * dz
    r2 = (d2x + d2y) + d2z

    lx = lmnT_ref[0:1, :]
    ly = lmnT_ref[1:2, :]
    lz = lmnT_ref[2:3, :]
    ax = jnp.where(lx == 0, 1.0, jnp.where(lx == 1, dx, d2x))
    ay = jnp.where(ly == 0, 1.0, jnp.where(ly == 1, dy, d2y))
    az = jnp.where(lz == 0, 1.0, jnp.where(lz == 1, dz, d2z))

    ex = jnp.exp2(at_ref[...] * r2)
    prim = (cn_ref[...] * ax) * (ay * az) * ex

    col = jax.lax.broadcasted_iota(jnp.int32, (NORB, NPRIM), 0)
    s = (col == oi_ref[...]).astype(jnp.float32)
    out_ref[...] = jax.lax.dot_general(
        prim, s, (((1,), (1,)), ((), ())),
        preferred_element_type=jnp.float32)


def _tc_part(pos, cn, centerT, at, lmnT, oi):
    grid = (NTC // BN,)
    return pl.pallas_call(
        _basis_block,
        grid=grid,
        in_specs=[
            pl.BlockSpec((BN, 3), lambda i: (i, 0)),
            pl.BlockSpec((1, NPRIM), lambda i: (0, 0)),
            pl.BlockSpec((3, NPRIM), lambda i: (0, 0)),
            pl.BlockSpec((1, NPRIM), lambda i: (0, 0)),
            pl.BlockSpec((3, NPRIM), lambda i: (0, 0)),
            pl.BlockSpec((1, NPRIM), lambda i: (0, 0)),
        ],
        out_specs=pl.BlockSpec((BN, NORB), lambda i: (i, 0)),
        out_shape=jax.ShapeDtypeStruct((NTC, NORB), jnp.float32),
        compiler_params=pltpu.CompilerParams(
            dimension_semantics=("parallel",)),
    )(pos, cn, centerT, at, lmnT, oi)


def _sc_body(pxr_hbm, pf_hbm, li_hbm, out_hbm, pxr_v, pf_v, li_v, prim_v):
    wid = lax.axis_index("s") * 2 + lax.axis_index("c")
    base = wid * RPW

    pltpu.sync_copy(pxr_hbm.at[pl.ds(base * 3 * L, RPW * 3 * L)], pxr_v)
    pltpu.sync_copy(pf_hbm, pf_v)
    pltpu.sync_copy(li_hbm, li_v)

    def row_body(r, _):
        x = pxr_v[pl.ds(r * 3 * L, L)]
        y = pxr_v[pl.ds(r * 3 * L + L, L)]
        z = pxr_v[pl.ds(r * 3 * L + 2 * L, L)]

        def slice_body(j, __):
            o = j * L
            cx = pf_v[pl.ds(o, L)]
            cy = pf_v[pl.ds(NPRIM + o, L)]
            cz = pf_v[pl.ds(2 * NPRIM + o, L)]
            at = pf_v[pl.ds(3 * NPRIM + o, L)]
            cn = pf_v[pl.ds(4 * NPRIM + o, L)]
            lx = li_v[pl.ds(o, L)]
            ly = li_v[pl.ds(NPRIM + o, L)]
            lz = li_v[pl.ds(2 * NPRIM + o, L)]
            dx = x - cx
            dy = y - cy
            dz = z - cz
            d2x = dx * dx
            d2y = dy * dy
            d2z = dz * dz
            r2 = (d2x + d2y) + d2z
            ax = jnp.where(lx == 0, 1.0, jnp.where(lx == 1, dx, d2x))
            ay = jnp.where(ly == 0, 1.0, jnp.where(ly == 1, dy, d2y))
            az = jnp.where(lz == 0, 1.0, jnp.where(lz == 1, dz, d2z))
            prim = (cn * ax) * (ay * az) * jnp.exp(at * r2)
            prim_v[pl.ds(r * NPRIM + o, L)] = prim
            return __

        lax.fori_loop(0, NPRIM // L, slice_body, 0, unroll=4)
        return _

    lax.fori_loop(0, RPW, row_body, 0, unroll=False)
    pltpu.sync_copy(prim_v, out_hbm.at[pl.ds(base * NPRIM, RPW * NPRIM)])


def _sc_part(pxr, pf, li):
    mesh = plsc.VectorSubcoreMesh(core_axis_name="c", subcore_axis_name="s")
    run = functools.partial(
        pl.kernel,
        mesh=mesh,
        out_type=jax.ShapeDtypeStruct((NSC * NPRIM,), jnp.float32),
        scratch_types=[
            pltpu.VMEM((RPW * 3 * L,), jnp.float32),
            pltpu.VMEM((5 * NPRIM,), jnp.float32),
            pltpu.VMEM((3 * NPRIM,), jnp.int32),
            pltpu.VMEM((RPW * NPRIM,), jnp.float32),
        ],
    )(_sc_body)
    return run(pxr, pf, li)


def _seg_block(prim_ref, oi_ref, out_ref):
    col = jax.lax.broadcasted_iota(jnp.int32, (NORB, NPRIM), 0)
    s = (col == oi_ref[...]).astype(jnp.float32)
    out_ref[...] = jax.lax.dot_general(
        prim_ref[...], s, (((1,), (1,)), ((), ())),
        preferred_element_type=jnp.float32)


def _seg_part(prim_sc, oi):
    return pl.pallas_call(
        _seg_block,
        grid=(NSC // BSC,),
        in_specs=[
            pl.BlockSpec((BSC, NPRIM), lambda i: (i, 0)),
            pl.BlockSpec((1, NPRIM), lambda i: (0, 0)),
        ],
        out_specs=pl.BlockSpec((BSC, NORB), lambda i: (i, 0)),
        out_shape=jax.ShapeDtypeStruct((NSC, NORB), jnp.float32),
        compiler_params=pltpu.CompilerParams(
            dimension_semantics=("parallel",)),
    )(prim_sc, oi)


@jax.jit
def kernel(pos, coefficients, center, alpha, norm, lmn, orbital_index):
    cn = (coefficients * norm).reshape(1, NPRIM)
    centerT = center.T
    lmnT = lmn.T
    at = (-_LOG2E * alpha).reshape(1, NPRIM)
    oi = orbital_index.reshape(1, NPRIM)

    out_tc = _tc_part(pos[:NTC], cn, centerT, at, lmnT, oi)

    pxr = jnp.repeat(pos[NTC:, :, None], L, axis=2).reshape(-1)
    pf = jnp.concatenate(
        [center[:, 0], center[:, 1], center[:, 2], -alpha,
         (coefficients * norm)], axis=0)
    li = lmnT.reshape(-1)
    prim_sc = _sc_part(pxr, pf, li).reshape(NSC, NPRIM)
    out_sc = _seg_part(prim_sc, oi)

    return jnp.concatenate([out_tc, out_sc], axis=0)


# speedup vs baseline: 2.4596x; 1.9833x over previous
import jax
import jax.numpy as jnp
from jax.experimental import pallas as pl
from jax.experimental.pallas import tpu as pltpu

NPOS = 8192
NPRIM = 1024
NORB = 256
BN = 1024

_LOG2E = 1.4426950408889634


def _basis_block(pos_ref, cn_ref, centerT_ref, at_ref, lmnT_ref, oi_ref,
                 out_ref, s_ref):
    @pl.when(pl.program_id(0) == 0)
    def _build_s():
        col = jax.lax.broadcasted_iota(jnp.int32, (NORB, NPRIM), 0)
        s_ref[...] = jnp.where(col == oi_ref[...], cn_ref[...], 0.0)

    p = pos_ref[...]
    x = p[:, 0:1]
    y = p[:, 1:2]
    z = p[:, 2:3]

    cx = centerT_ref[0:1, :]
    cy = centerT_ref[1:2, :]
    cz = centerT_ref[2:3, :]

    dx = x - cx
    dy = y - cy
    dz = z - cz
    d2x = dx * dx
    d2y = dy * dy
    d2z = dz * dz
    r2 = (d2x + d2y) + d2z

    lx = lmnT_ref[0:1, :]
    ly = lmnT_ref[1:2, :]
    lz = lmnT_ref[2:3, :]
    ax = jnp.where(lx == 0, 1.0, jnp.where(lx == 1, dx, d2x))
    ay = jnp.where(ly == 0, 1.0, jnp.where(ly == 1, dy, d2y))
    az = jnp.where(lz == 0, 1.0, jnp.where(lz == 1, dz, d2z))

    ex = jnp.exp2(at_ref[...] * r2)
    prim = (ax * ay) * (az * ex)

    out_ref[...] = jax.lax.dot_general(
        prim, s_ref[...], (((1,), (1,)), ((), ())),
        preferred_element_type=jnp.float32)


@jax.jit
def kernel(pos, coefficients, center, alpha, norm, lmn, orbital_index):
    cn = (coefficients * norm).reshape(1, NPRIM)
    centerT = center.T
    lmnT = lmn.T
    at = (-_LOG2E * alpha).reshape(1, NPRIM)
    oi = orbital_index.reshape(1, NPRIM)

    grid = (NPOS // BN,)
    return pl.pallas_call(
        _basis_block,
        grid=grid,
        in_specs=[
            pl.BlockSpec((BN, 3), lambda i: (i, 0)),
            pl.BlockSpec((1, NPRIM), lambda i: (0, 0)),
            pl.BlockSpec((3, NPRIM), lambda i: (0, 0)),
            pl.BlockSpec((1, NPRIM), lambda i: (0, 0)),
            pl.BlockSpec((3, NPRIM), lambda i: (0, 0)),
            pl.BlockSpec((1, NPRIM), lambda i: (0, 0)),
        ],
        out_specs=pl.BlockSpec((BN, NORB), lambda i: (i, 0)),
        out_shape=jax.ShapeDtypeStruct((NPOS, NORB), jnp.float32),
        scratch_shapes=[pltpu.VMEM((NORB, NPRIM), jnp.float32)],
        compiler_params=pltpu.CompilerParams(
            dimension_semantics=("arbitrary",)),
    )(pos, cn, centerT, at, lmnT, oi)
